# Initial kernel scaffold; baseline (speedup 1.0000x reference)
import jax
import jax.numpy as jnp
from jax.experimental import pallas as pl


def kernel(output_probs, target_class):
    raise NotImplementedError("write your pallas kernel here")


# speedup vs baseline: 1.6425x; 1.6425x over previous
import jax
import jax.numpy as jnp
from jax.experimental import pallas as pl
from jax.experimental.pallas import tpu as pltpu

_N_CLS = 1000
_ROWS = 16384
_BR = 1024
_GRID = _ROWS // _BR
_SCALE = 1.0 / (_ROWS * _N_CLS)


def _body(t_ref, p_ref, o_ref):
    i = pl.program_id(0)
    t = t_ref[...]
    j = jax.lax.broadcasted_iota(jnp.int32, (_BR, _N_CLS), 1).astype(jnp.float32)
    w = jnp.abs(j - t) * jnp.float32(_SCALE)
    partial = jnp.sum(w * p_ref[...])

    @pl.when(i == 0)
    def _init():
        o_ref[0, 0] = 0.0

    o_ref[0, 0] += partial


def kernel(output_probs, target_class):
    t = target_class.astype(jnp.float32).reshape(_ROWS, 1)
    out = pl.pallas_call(
        _body,
        grid=(_GRID,),
        in_specs=[
            pl.BlockSpec((_BR, 1), lambda i: (i, 0)),
            pl.BlockSpec((_BR, _N_CLS), lambda i: (i, 0)),
        ],
        out_specs=pl.BlockSpec(memory_space=pltpu.SMEM),
        out_shape=jax.ShapeDtypeStruct((1, 1), jnp.float32),
    )(t, output_probs)
    return out[0, 0]
